# baseline (speedup 1.0000x reference)
import functools

import jax
import jax.numpy as jnp
from jax.experimental import pallas as pl
from jax.experimental.pallas import tpu as pltpu

_LANES = 128
_SPLIT = 8
_NRING = 4
_NOUT = 2


def _block_partials_ref(ring, slot, r_rows):
    sr = r_rows // _SPLIT
    m = None
    s = None
    for i in range(_SPLIT):
        sub = ring[slot, pl.ds(i * sr, sr), :]
        mi = jnp.max(sub, axis=0, keepdims=True)
        si = jnp.sum(jnp.exp(sub - mi), axis=0, keepdims=True)
        if m is None:
            m, s = mi, si
        else:
            mn = jnp.maximum(m, mi)
            s = s * jnp.exp(m - mn) + si * jnp.exp(mi - mn)
            m = mn
    return m, s


def _manual_kernel(nblk, kc, x_hbm, o_hbm, in_ring, out_ring, cache,
                   in_sems, out_sems):
    neg_big = jnp.float32(-3.4e38)

    def rd(k, slot):
        return pltpu.make_async_copy(x_hbm.at[k], in_ring.at[slot],
                                     in_sems.at[slot])

    def wr(slot, k):
        return pltpu.make_async_copy(out_ring.at[slot], o_hbm.at[k],
                                     out_sems.at[slot])

    for j in range(_NRING):
        rd(j, j).start()

    def a_body(k, carry):
        m_run, s_run = carry
        slot = jax.lax.rem(k, _NRING)
        rd(k, slot).wait()
        m_blk, s_blk = _block_partials_ref(in_ring, slot, in_ring.shape[1])

        @pl.when(k < kc)
        def _():
            cache[pl.ds(jnp.minimum(k, kc - 1), 1)] = (
                in_ring[slot].astype(jnp.bfloat16)[None])

        @pl.when(k + _NRING < nblk)
        def _():
            rd(k + _NRING, slot).start()

        m_new = jnp.maximum(m_run, m_blk)
        s_new = (s_run * jnp.exp(m_run - m_new)
                 + s_blk * jnp.exp(m_blk - m_new))
        return m_new, s_new

    m0 = jnp.full((1, _LANES), neg_big, dtype=jnp.float32)
    s0 = jnp.zeros((1, _LANES), dtype=jnp.float32)
    m_run, s_run = jax.lax.fori_loop(0, nblk, a_body, (m0, s0))

    m_gl = jnp.max(m_run, axis=1, keepdims=True)
    s_gl = jnp.sum(s_run * jnp.exp(m_run - m_gl),
                   axis=1, keepdims=True)
    r = 1.0 / s_gl

    def b1_body(k, _):
        oslot = jax.lax.rem(k, _NOUT)

        @pl.when(k >= _NOUT)
        def _():
            wr(oslot, k).wait()

        kk = jnp.minimum(k, kc - 1)
        sr = out_ring.shape[1] // _SPLIT
        for i in range(_SPLIT):
            out_ring[oslot, pl.ds(i * sr, sr), :] = jnp.exp(
                cache[kk, pl.ds(i * sr, sr), :].astype(jnp.float32)
                - m_gl) * r
        wr(oslot, k).start()
        return 0

    jax.lax.fori_loop(0, kc, b1_body, 0)

    for j in range(_NRING):
        if kc + j < nblk:
            rd(kc + j, j).start()

    def b2_body(k, _):
        islot = jax.lax.rem(k - kc, _NRING)
        oslot = jax.lax.rem(k, _NOUT)
        rd(k, islot).wait()

        @pl.when(k >= _NOUT)
        def _():
            wr(oslot, k).wait()

        sr = out_ring.shape[1] // _SPLIT
        for i in range(_SPLIT):
            out_ring[oslot, pl.ds(i * sr, sr), :] = jnp.exp(
                in_ring[islot, pl.ds(i * sr, sr), :] - m_gl) * r
        wr(oslot, k).start()

        @pl.when(k + _NRING < nblk)
        def _():
            rd(k + _NRING, islot).start()

        return 0

    jax.lax.fori_loop(kc, nblk, b2_body, 0)

    for j in range(_NOUT):
        k_last = nblk - _NOUT + j
        if k_last >= 0:
            wr(k_last % _NOUT, k_last).wait()


def _softmax_manual(x, nblk, kc):
    n = x.shape[0]
    rows = n // _LANES
    r_rows = rows // nblk
    x3 = x.reshape(nblk, r_rows, _LANES)

    out3 = pl.pallas_call(
        functools.partial(_manual_kernel, nblk, kc),
        out_shape=jax.ShapeDtypeStruct((nblk, r_rows, _LANES), jnp.float32),
        in_specs=[pl.BlockSpec(memory_space=pl.ANY)],
        out_specs=pl.BlockSpec(memory_space=pl.ANY),
        scratch_shapes=[
            pltpu.VMEM((_NRING, r_rows, _LANES), jnp.float32),
            pltpu.VMEM((_NOUT, r_rows, _LANES), jnp.float32),
            pltpu.VMEM((kc, r_rows, _LANES), jnp.bfloat16),
            pltpu.SemaphoreType.DMA((_NRING,)),
            pltpu.SemaphoreType.DMA((_NOUT,)),
        ],
        compiler_params=pltpu.CompilerParams(
            vmem_limit_bytes=64 * 1024 * 1024,
            internal_scratch_in_bytes=1024 * 1024),
        name="softmax_manual",
    )(x3)

    return out3.reshape(n)


def kernel(x):
    return _softmax_manual(x, nblk=64, kc=50)


# speedup vs baseline: 1.1030x; 1.1030x over previous
import functools

import jax
import jax.numpy as jnp
from jax.experimental import pallas as pl
from jax.experimental.pallas import tpu as pltpu

_LANES = 128
_SPLIT = 8
_NRING = 4
_NOUT = 3


def _block_partials_ref(ring, slot, r_rows):
    sr = r_rows // _SPLIT
    m = None
    s = None
    for i in range(_SPLIT):
        sub = ring[slot, pl.ds(i * sr, sr), :]
        mi = jnp.max(sub, axis=0, keepdims=True)
        si = jnp.sum(jnp.exp(sub - mi), axis=0, keepdims=True)
        if m is None:
            m, s = mi, si
        else:
            mn = jnp.maximum(m, mi)
            s = s * jnp.exp(m - mn) + si * jnp.exp(mi - mn)
            m = mn
    return m, s


def _manual_kernel(nblk, kc, x_hbm, o_hbm, in_ring, out_ring, cache,
                   in_sems, out_sems):
    neg_big = jnp.float32(-3.4e38)

    def rd(k, slot):
        return pltpu.make_async_copy(x_hbm.at[k], in_ring.at[slot],
                                     in_sems.at[slot])

    def wr(slot, k):
        return pltpu.make_async_copy(out_ring.at[slot], o_hbm.at[k],
                                     out_sems.at[slot])

    for j in range(_NRING):
        rd(j, j).start()

    def a_body(k, carry):
        m_run, s_run = carry
        slot = jax.lax.rem(k, _NRING)
        rd(k, slot).wait()
        m_blk, s_blk = _block_partials_ref(in_ring, slot, in_ring.shape[1])

        @pl.when(k < kc)
        def _():
            cache[pl.ds(jnp.minimum(k, kc - 1), 1)] = (
                in_ring[slot].astype(jnp.bfloat16)[None])

        @pl.when(k + _NRING < nblk)
        def _():
            rd(k + _NRING, slot).start()

        m_new = jnp.maximum(m_run, m_blk)
        s_new = (s_run * jnp.exp(m_run - m_new)
                 + s_blk * jnp.exp(m_blk - m_new))
        return m_new, s_new

    m0 = jnp.full((1, _LANES), neg_big, dtype=jnp.float32)
    s0 = jnp.zeros((1, _LANES), dtype=jnp.float32)
    m_run, s_run = jax.lax.fori_loop(0, nblk, a_body, (m0, s0))

    m_gl = jnp.max(m_run, axis=1, keepdims=True)
    s_gl = jnp.sum(s_run * jnp.exp(m_run - m_gl),
                   axis=1, keepdims=True)
    r = 1.0 / s_gl

    def b1_body(k, _):
        oslot = jax.lax.rem(k, _NOUT)

        @pl.when(k >= _NOUT)
        def _():
            wr(oslot, k).wait()

        kk = jnp.minimum(k, kc - 1)
        sr = out_ring.shape[1] // _SPLIT
        for i in range(_SPLIT):
            out_ring[oslot, pl.ds(i * sr, sr), :] = jnp.exp(
                cache[kk, pl.ds(i * sr, sr), :].astype(jnp.float32)
                - m_gl) * r
        wr(oslot, k).start()
        return 0

    jax.lax.fori_loop(0, kc, b1_body, 0)

    for j in range(_NRING):
        if kc + j < nblk:
            rd(kc + j, j).start()

    def b2_body(k, _):
        islot = jax.lax.rem(k - kc, _NRING)
        oslot = jax.lax.rem(k, _NOUT)
        rd(k, islot).wait()

        @pl.when(k >= _NOUT)
        def _():
            wr(oslot, k).wait()

        sr = out_ring.shape[1] // _SPLIT
        for i in range(_SPLIT):
            out_ring[oslot, pl.ds(i * sr, sr), :] = jnp.exp(
                in_ring[islot, pl.ds(i * sr, sr), :] - m_gl) * r
        wr(oslot, k).start()

        @pl.when(k + _NRING < nblk)
        def _():
            rd(k + _NRING, islot).start()

        return 0

    jax.lax.fori_loop(kc, nblk, b2_body, 0)

    for j in range(_NOUT):
        k_last = nblk - _NOUT + j
        if k_last >= 0:
            wr(k_last % _NOUT, k_last).wait()


def _softmax_manual(x, nblk, kc):
    n = x.shape[0]
    rows = n // _LANES
    r_rows = rows // nblk
    x3 = x.reshape(nblk, r_rows, _LANES)

    out3 = pl.pallas_call(
        functools.partial(_manual_kernel, nblk, kc),
        out_shape=jax.ShapeDtypeStruct((nblk, r_rows, _LANES), jnp.float32),
        in_specs=[pl.BlockSpec(memory_space=pl.ANY)],
        out_specs=pl.BlockSpec(memory_space=pl.ANY),
        scratch_shapes=[
            pltpu.VMEM((_NRING, r_rows, _LANES), jnp.float32),
            pltpu.VMEM((_NOUT, r_rows, _LANES), jnp.float32),
            pltpu.VMEM((kc, r_rows, _LANES), jnp.bfloat16),
            pltpu.SemaphoreType.DMA((_NRING,)),
            pltpu.SemaphoreType.DMA((_NOUT,)),
        ],
        compiler_params=pltpu.CompilerParams(
            vmem_limit_bytes=64 * 1024 * 1024,
            internal_scratch_in_bytes=1024 * 1024),
        name="softmax_manual",
    )(x3)

    return out3.reshape(n)


def kernel(x):
    return _softmax_manual(x, nblk=64, kc=49)
